# Initial kernel scaffold; baseline (speedup 1.0000x reference)
import jax
import jax.numpy as jnp
from jax.experimental import pallas as pl


def kernel(t, idx):
    raise NotImplementedError("write your pallas kernel here")


# speedup vs baseline: 1.4990x; 1.4990x over previous
import functools

import jax
import jax.numpy as jnp
from jax import lax
from jax.experimental import pallas as pl
from jax.experimental.pallas import tpu as pltpu
from jax.experimental.pallas import tpu_sc as plsc

NC = 2
NS = 16
NW = NC * NS
D = 32
G = 128

B = 16384 * 20
B_PER_W = B // NW
GROUPS_PER_W = B_PER_W // G
CHUNK_GROUPS = 8
CHUNK_ROWS = CHUNK_GROUPS * G
NCHUNKS = GROUPS_PER_W // CHUNK_GROUPS

_mesh = plsc.VectorSubcoreMesh(core_axis_name="c", subcore_axis_name="s")


@functools.partial(
    pl.kernel,
    mesh=_mesh,
    out_type=jax.ShapeDtypeStruct((B, D), jnp.float32),
    scratch_types=[
        pltpu.VMEM((GROUPS_PER_W, G), jnp.int32),
        pltpu.VMEM((CHUNK_ROWS, D), jnp.float32),
        pltpu.SemaphoreType.DMA,
    ],
    compiler_params=pltpu.CompilerParams(use_tc_tiling_on_sc=False),
)
def _gather_kernel(t_hbm, idx_hbm, out_hbm, idx_v, buf, sem):
    wid = lax.axis_index("s") * NC + lax.axis_index("c")
    row0 = wid * B_PER_W
    pltpu.sync_copy(idx_hbm.at[pl.ds(wid * GROUPS_PER_W, GROUPS_PER_W)], idx_v)

    def chunk_body(c, carry):
        copies = []
        for j in range(CHUNK_GROUPS):
            copies.append(
                pltpu.async_copy(
                    t_hbm.at[idx_v.at[c * CHUNK_GROUPS + j]],
                    buf.at[pl.ds(j * G, G)],
                    sem,
                )
            )
        for cp in copies:
            cp.wait()
        pltpu.sync_copy(buf, out_hbm.at[pl.ds(row0 + c * CHUNK_ROWS, CHUNK_ROWS)])
        return carry

    lax.fori_loop(0, NCHUNKS, chunk_body, 0)


def kernel(t, idx):
    idx32 = idx.astype(jnp.int32).reshape(B // G, G)
    out = _gather_kernel(t, idx32)
    return out.reshape(idx.shape[0], idx.shape[1], D)


# speedup vs baseline: 1.5130x; 1.0093x over previous
import functools

import jax
import jax.numpy as jnp
from jax import lax
from jax.experimental import pallas as pl
from jax.experimental.pallas import tpu as pltpu
from jax.experimental.pallas import tpu_sc as plsc

NC = 2
NS = 16
NW = NC * NS
D = 32
G = 128

B = 16384 * 20
B_PER_W = B // NW
GROUPS_PER_W = B_PER_W // G
CHUNK_GROUPS = 10
CHUNK_ROWS = CHUNK_GROUPS * G
NCHUNKS = GROUPS_PER_W // CHUNK_GROUPS

_mesh = plsc.VectorSubcoreMesh(core_axis_name="c", subcore_axis_name="s")


@functools.partial(
    pl.kernel,
    mesh=_mesh,
    out_type=jax.ShapeDtypeStruct((B, D), jnp.float32),
    scratch_types=[
        pltpu.VMEM((GROUPS_PER_W, G), jnp.int32),
        pltpu.VMEM((CHUNK_ROWS, D), jnp.float32),
        pltpu.VMEM((CHUNK_ROWS, D), jnp.float32),
        pltpu.SemaphoreType.DMA,
        pltpu.SemaphoreType.DMA,
        pltpu.SemaphoreType.DMA,
        pltpu.SemaphoreType.DMA,
    ],
    compiler_params=pltpu.CompilerParams(use_tc_tiling_on_sc=False),
)
def _gather_kernel(t_hbm, idx_hbm, out_hbm, idx_v, buf0, buf1,
                   gsem0, gsem1, osem0, osem1):
    wid = lax.axis_index("s") * NC + lax.axis_index("c")
    row0 = wid * B_PER_W
    bufs = (buf0, buf1)
    gsems = (gsem0, gsem1)
    osems = (osem0, osem1)

    pltpu.sync_copy(idx_hbm.at[pl.ds(wid * GROUPS_PER_W, GROUPS_PER_W)], idx_v)

    def fire_gathers(c, b):
        for j in range(CHUNK_GROUPS):
            pltpu.async_copy(
                t_hbm.at[idx_v.at[c * CHUNK_GROUPS + j]],
                bufs[b].at[pl.ds(j * G, G)],
                gsems[b],
            )

    def drain_gathers(b):
        for j in range(CHUNK_GROUPS):
            pltpu.make_async_copy(
                t_hbm.at[pl.ds(0, G)],
                bufs[b].at[pl.ds(j * G, G)],
                gsems[b],
            ).wait()

    def write_out(c, b):
        cp = pltpu.async_copy(
            bufs[b], out_hbm.at[pl.ds(row0 + c * CHUNK_ROWS, CHUNK_ROWS)],
            osems[b],
        )
        return cp

    fire_gathers(0, 0)
    fire_gathers(1, 1)

    def pair_body(c2, carry):
        for b in range(2):
            c = 2 * c2 + b
            drain_gathers(b)
            cp = write_out(c, b)
            cp.wait()
            fire_gathers(c + 2, b)
        return carry

    lax.fori_loop(0, (NCHUNKS - 2) // 2, pair_body, 0)

    for c in (NCHUNKS - 2, NCHUNKS - 1):
        b = c % 2
        drain_gathers(b)
        write_out(c, b).wait()


def kernel(t, idx):
    idx32 = idx.astype(jnp.int32).reshape(B // G, G)
    out = _gather_kernel(t, idx32)
    return out.reshape(idx.shape[0], idx.shape[1], D)
